# baseline (speedup 1.0000x reference)
import jax
import jax.numpy as jnp
from jax.experimental import pallas as pl
from jax.experimental.pallas import tpu as pltpu

CODEBOOK_SIZE = 1024
CODEBOOK_DIM = 64
COMMITMENT_COST = 0.25

TILE_N = 512


def _vq_kernel(x_ref, c_ref, q_ref, idx_ref, loss_ref, acc_ref):
    i = pl.program_id(0)
    nsteps = pl.num_programs(0)

    x = x_ref[...]
    c = c_ref[...]

    x2 = jnp.sum(x * x, axis=1, keepdims=True)
    c2 = jnp.sum(c * c, axis=1)
    xc = jax.lax.dot_general(
        x, c, (((1,), (1,)), ((), ())),
        preferred_element_type=jnp.float32)
    d = x2 + c2[None, :] - 2.0 * xc

    dmin = jnp.min(d, axis=1, keepdims=True)
    iota = jax.lax.broadcasted_iota(jnp.int32, d.shape, 1)
    idx = jnp.min(jnp.where(d == dmin, iota, CODEBOOK_SIZE), axis=1)
    idx_ref[...] = idx.astype(jnp.int32)

    onehot = (iota == idx[:, None]).astype(jnp.float32)
    q = jax.lax.dot_general(
        onehot, c, (((1,), (0,)), ((), ())),
        preferred_element_type=jnp.float32,
        precision=jax.lax.Precision.HIGHEST)
    q_ref[...] = q

    diff = q - x
    part = jnp.sum(diff * diff)

    @pl.when(i == 0)
    def _():
        acc_ref[0, 0] = part

    @pl.when(i != 0)
    def _():
        acc_ref[0, 0] = acc_ref[0, 0] + part

    @pl.when(i == nsteps - 1)
    def _():
        total_elems = nsteps * TILE_N * CODEBOOK_DIM
        loss_ref[0, 0] = acc_ref[0, 0] * (COMMITMENT_COST / total_elems)


def kernel(inputs, codebook):
    batch, time_steps, dim = inputs.shape
    n = batch * time_steps
    flat = inputs.reshape(n, dim)
    grid = n // TILE_N

    q, idx, loss = pl.pallas_call(
        _vq_kernel,
        grid=(grid,),
        in_specs=[
            pl.BlockSpec((TILE_N, dim), lambda i: (i, 0)),
            pl.BlockSpec((CODEBOOK_SIZE, dim), lambda i: (0, 0)),
        ],
        out_specs=[
            pl.BlockSpec((TILE_N, dim), lambda i: (i, 0)),
            pl.BlockSpec((TILE_N,), lambda i: (i,)),
            pl.BlockSpec(memory_space=pltpu.SMEM),
        ],
        out_shape=[
            jax.ShapeDtypeStruct((n, dim), jnp.float32),
            jax.ShapeDtypeStruct((n,), jnp.int32),
            jax.ShapeDtypeStruct((1, 1), jnp.float32),
        ],
        scratch_shapes=[pltpu.SMEM((1, 1), jnp.float32)],
    )(flat, codebook)

    quantized = q.reshape(batch, time_steps, dim)
    indices = idx.reshape(batch, time_steps)
    return quantized, indices, loss[0, 0]


# speedup vs baseline: 1.6017x; 1.6017x over previous
import jax
import jax.numpy as jnp
from jax import lax
from jax.experimental import pallas as pl
from jax.experimental.pallas import tpu as pltpu
from jax.experimental.pallas import tpu_sc as plsc

CODEBOOK_SIZE = 1024
CODEBOOK_DIM = 64
COMMITMENT_COST = 0.25

TILE_N = 512

NUM_WORKERS = 32
IDX_CHUNK = 128


def _argmin_kernel(x_ref, c_ref, idx_ref, loss_ref, acc_ref):
    i = pl.program_id(0)
    nsteps = pl.num_programs(0)

    x = x_ref[...]
    c = c_ref[...]

    x2 = jnp.sum(x * x, axis=1, keepdims=True)
    c2 = jnp.sum(c * c, axis=1)
    xc = jax.lax.dot_general(
        x, c, (((1,), (1,)), ((), ())),
        preferred_element_type=jnp.float32)
    d = x2 + c2[None, :] - 2.0 * xc

    dmin = jnp.min(d, axis=1, keepdims=True)
    iota = jax.lax.broadcasted_iota(jnp.int32, d.shape, 1)
    idx = jnp.min(jnp.where(d == dmin, iota, CODEBOOK_SIZE), axis=1)
    idx_ref[...] = idx.astype(jnp.int32)

    part = jnp.sum(dmin)

    @pl.when(i == 0)
    def _():
        acc_ref[0, 0] = part

    @pl.when(i != 0)
    def _():
        acc_ref[0, 0] = acc_ref[0, 0] + part

    @pl.when(i == nsteps - 1)
    def _():
        total_elems = nsteps * TILE_N * CODEBOOK_DIM
        loss_ref[0, 0] = acc_ref[0, 0] * (COMMITMENT_COST / total_elems)


def _make_sc_gather(n_rows):
    rows_per_w = n_rows // NUM_WORKERS
    n_chunks = rows_per_w // IDX_CHUNK
    mesh = plsc.VectorSubcoreMesh(core_axis_name="c", subcore_axis_name="s")

    def sc_gather(codebook, idx_grid):
        @pl.kernel(
            mesh=mesh,
            out_type=jax.ShapeDtypeStruct((n_rows, CODEBOOK_DIM), jnp.float32),
            scratch_types=[
                pltpu.VMEM((n_chunks, IDX_CHUNK), jnp.int32),
                pltpu.VMEM((rows_per_w, CODEBOOK_DIM), jnp.float32),
                pltpu.SemaphoreType.DMA,
            ],
            compiler_params=pltpu.CompilerParams(use_tc_tiling_on_sc=False),
        )
        def body(cb_hbm, idx_hbm, out_hbm, idx_v, rows_v, sem):
            wid = lax.axis_index("s") * 2 + lax.axis_index("c")
            base = wid * rows_per_w
            pltpu.sync_copy(idx_hbm.at[wid], idx_v)
            copies = [
                pltpu.async_copy(
                    cb_hbm.at[idx_v.at[j]],
                    rows_v.at[pl.ds(j * IDX_CHUNK, IDX_CHUNK)],
                    sem)
                for j in range(n_chunks)
            ]
            for cp in copies:
                cp.wait()
            pltpu.sync_copy(rows_v, out_hbm.at[pl.ds(base, rows_per_w)])

        return body(codebook, idx_grid)

    return sc_gather


def kernel(inputs, codebook):
    batch, time_steps, dim = inputs.shape
    n = batch * time_steps
    flat = inputs.reshape(n, dim)
    grid = n // TILE_N

    idx, loss = pl.pallas_call(
        _argmin_kernel,
        grid=(grid,),
        in_specs=[
            pl.BlockSpec((TILE_N, dim), lambda i: (i, 0)),
            pl.BlockSpec((CODEBOOK_SIZE, dim), lambda i: (0, 0)),
        ],
        out_specs=[
            pl.BlockSpec((TILE_N,), lambda i: (i,)),
            pl.BlockSpec(memory_space=pltpu.SMEM),
        ],
        out_shape=[
            jax.ShapeDtypeStruct((n,), jnp.int32),
            jax.ShapeDtypeStruct((1, 1), jnp.float32),
        ],
        scratch_shapes=[pltpu.SMEM((1, 1), jnp.float32)],
    )(flat, codebook)

    idx_grid = idx.reshape(NUM_WORKERS, n // (NUM_WORKERS * IDX_CHUNK), IDX_CHUNK)
    q = _make_sc_gather(n)(codebook, idx_grid)

    quantized = q.reshape(batch, time_steps, dim)
    indices = idx.reshape(batch, time_steps)
    return quantized, indices, loss[0, 0]
